# baseline (speedup 1.0000x reference)
import functools

import jax
import jax.numpy as jnp
import numpy as np
from jax.experimental import pallas as pl
from jax.experimental.pallas import tpu as pltpu

_K_OUT = 1024
_Q_BLK = 512


def _fused_kernel(x_ref, xq_ref, lab_ref, labq_ref, wq_ref, bq_ref, wk_ref, bk_ref, wv_ref,
                  bv_ref, wo_ref, bo_ref, w1_ref, b1_ref, w2_ref, b2_ref,
                  out_ref, *, scale, q_blk):
    x = x_ref[...]
    n = x.shape[0]
    xq = xq_ref[...]

    q = jnp.dot(xq, wq_ref[...] * scale,
                preferred_element_type=jnp.float32) + bq_ref[...] * scale
    k = jnp.dot(x, wk_ref[...], preferred_element_type=jnp.float32) + bk_ref[...]
    v3 = jnp.dot(x, wv_ref[...], preferred_element_type=jnp.float32) + bv_ref[...]
    v = jnp.concatenate([v3, jnp.ones((n, 1), jnp.float32)], axis=1)

    s = jax.lax.dot_general(q, k, (((1,), (1,)), ((), ())),
                            preferred_element_type=jnp.float32)

    lab = lab_ref[...]
    lq = jnp.reshape(labq_ref[...], (q_blk, 1))
    lq = jnp.where(lq == -1, -2, lq)
    e = jnp.exp2(jnp.where(lq == lab, s, -jnp.inf))

    oa = jnp.dot(e, v, preferred_element_type=jnp.float32)
    den = oa[:, 3:4]
    o = oa[:, :3] * jnp.where(den > 0, 1.0 / den, 0.0)

    w1f = jnp.dot(wo_ref[...], w1_ref[...], preferred_element_type=jnp.float32)
    b1f = jnp.dot(bo_ref[...], w1_ref[...],
                  preferred_element_type=jnp.float32) + b1_ref[...]
    h = jnp.maximum(
        jnp.dot(o, w1f, preferred_element_type=jnp.float32) + b1f, 0.0)
    out_ref[...] = (
        jnp.dot(h, w2_ref[...], preferred_element_type=jnp.float32)
        + b2_ref[...])


def kernel(x, Wq, bq, Wk, bk, Wv, bv, Wo, bo, W1, b1, W2, b2, labels):
    B, N, D = x.shape
    H = W1.shape[1]
    scale = float(np.log2(np.e) / np.sqrt(D))
    lab = labels.astype(jnp.int32).reshape(B, 1, N)

    wspec = lambda shape: pl.BlockSpec(shape, lambda b, qi: (0, 0))
    out = pl.pallas_call(
        functools.partial(_fused_kernel, scale=scale, q_blk=_Q_BLK),
        grid=(B, _K_OUT // _Q_BLK),
        in_specs=[
            pl.BlockSpec((None, N, D), lambda b, qi: (b, 0, 0)),
            pl.BlockSpec((None, _Q_BLK, D), lambda b, qi: (b, qi, 0)),
            pl.BlockSpec((None, 1, N), lambda b, qi: (b, 0, 0)),
            pl.BlockSpec((None, 1, _Q_BLK), lambda b, qi: (b, 0, qi)),
            wspec((D, D)), wspec((1, D)),
            wspec((D, D)), wspec((1, D)),
            wspec((D, D)), wspec((1, D)),
            wspec((D, D)), wspec((1, D)),
            wspec((D, H)), wspec((1, H)),
            wspec((H, D)), wspec((1, D)),
        ],
        out_specs=pl.BlockSpec((None, _Q_BLK, D), lambda b, qi: (b, qi, 0)),
        out_shape=jax.ShapeDtypeStruct((B, _K_OUT, D), jnp.float32),
        compiler_params=pltpu.CompilerParams(
            dimension_semantics=("arbitrary", "arbitrary")),
    )(x, x, lab, lab, Wq, bq.reshape(1, D), Wk, bk.reshape(1, D), Wv,
      bv.reshape(1, D), Wo, bo.reshape(1, D), W1, b1.reshape(1, H),
      W2, b2.reshape(1, D))
    return out


# speedup vs baseline: 1.0956x; 1.0956x over previous
import functools

import jax
import jax.numpy as jnp
import numpy as np
from jax.experimental import pallas as pl
from jax.experimental.pallas import tpu as pltpu

_K_OUT = 1024


def _fused_kernel(x_ref, lab_ref, wq_ref, bq_ref, wk_ref, bk_ref, wv_ref,
                  bv_ref, wo_ref, bo_ref, w1_ref, b1_ref, w2_ref, b2_ref,
                  out_ref, *, scale, k_out):
    x = x_ref[...]
    n = x.shape[0]
    xq = x[:k_out]

    q = jnp.dot(xq, wq_ref[...] * scale,
                preferred_element_type=jnp.float32) + bq_ref[...] * scale
    k = jnp.dot(x, wk_ref[...], preferred_element_type=jnp.float32) + bk_ref[...]
    v3 = jnp.dot(x, wv_ref[...], preferred_element_type=jnp.float32) + bv_ref[...]
    v = jnp.concatenate([v3, jnp.ones((n, 1), jnp.float32)], axis=1)

    s = jax.lax.dot_general(q, k, (((1,), (1,)), ((), ())),
                            preferred_element_type=jnp.float32)

    lab = lab_ref[...]
    lq = jnp.reshape(lab[:, :k_out], (k_out, 1))
    lq = jnp.where(lq == -1, -2, lq)
    e = jnp.exp2(jnp.where(lq == lab, s, -jnp.inf))

    oa = jnp.dot(e.astype(jnp.bfloat16), v.astype(jnp.bfloat16),
                 preferred_element_type=jnp.float32)
    den = oa[:, 3:4]
    o = oa[:, :3] * jnp.where(den > 0, 1.0 / den, 0.0)

    w1f = jnp.dot(wo_ref[...], w1_ref[...], preferred_element_type=jnp.float32)
    b1f = jnp.dot(bo_ref[...], w1_ref[...],
                  preferred_element_type=jnp.float32) + b1_ref[...]
    h = jnp.maximum(
        jnp.dot(o, w1f, preferred_element_type=jnp.float32) + b1f, 0.0)
    out_ref[...] = (
        jnp.dot(h, w2_ref[...], preferred_element_type=jnp.float32)
        + b2_ref[...])


def kernel(x, Wq, bq, Wk, bk, Wv, bv, Wo, bo, W1, b1, W2, b2, labels):
    B, N, D = x.shape
    H = W1.shape[1]
    scale = float(np.log2(np.e) / np.sqrt(D))
    lab = labels.astype(jnp.int32).reshape(B, 1, N)

    wspec = lambda shape: pl.BlockSpec(shape, lambda b: (0, 0))
    out = pl.pallas_call(
        functools.partial(_fused_kernel, scale=scale, k_out=_K_OUT),
        grid=(B,),
        in_specs=[
            pl.BlockSpec((None, N, D), lambda b: (b, 0, 0)),
            pl.BlockSpec((None, 1, N), lambda b: (b, 0, 0)),
            wspec((D, D)), wspec((1, D)),
            wspec((D, D)), wspec((1, D)),
            wspec((D, D)), wspec((1, D)),
            wspec((D, D)), wspec((1, D)),
            wspec((D, H)), wspec((1, H)),
            wspec((H, D)), wspec((1, D)),
        ],
        out_specs=pl.BlockSpec((None, _K_OUT, D), lambda b: (b, 0, 0)),
        out_shape=jax.ShapeDtypeStruct((B, _K_OUT, D), jnp.float32),
        compiler_params=pltpu.CompilerParams(
            dimension_semantics=("arbitrary",)),
    )(x, lab, Wq, bq.reshape(1, D), Wk, bk.reshape(1, D), Wv,
      bv.reshape(1, D), Wo, bo.reshape(1, D), W1, b1.reshape(1, H),
      W2, b2.reshape(1, D))
    return out


# speedup vs baseline: 1.1015x; 1.0053x over previous
import functools

import jax
import jax.numpy as jnp
import numpy as np
from jax.experimental import pallas as pl
from jax.experimental.pallas import tpu as pltpu

_K_OUT = 1024


def _fused_kernel(x_ref, lab_ref, wq_ref, bq_ref, wk_ref, bk_ref, wv_ref,
                  bv_ref, wo_ref, bo_ref, w1_ref, b1_ref, w2_ref, b2_ref,
                  out_ref, *, scale, k_out):
    x = x_ref[...]
    n = x.shape[0]
    xq = x[:k_out]

    q = jnp.dot(xq, wq_ref[...] * scale,
                preferred_element_type=jnp.float32) + bq_ref[...] * scale
    k = jnp.dot(x, wk_ref[...], preferred_element_type=jnp.float32) + bk_ref[...]
    v3 = jnp.dot(x, wv_ref[...], preferred_element_type=jnp.float32) + bv_ref[...]
    v = jnp.concatenate([v3, jnp.ones((n, 1), jnp.float32)], axis=1)

    s = jax.lax.dot_general(q, k, (((1,), (1,)), ((), ())),
                            preferred_element_type=jnp.float32)

    lab = lab_ref[...]
    lq = jnp.reshape(lab[:, :k_out], (k_out, 1))
    lq = jnp.where(lq == -1, -2, lq)
    e = jnp.exp2(jnp.where(lq == lab, s, -jnp.inf))

    oa = jnp.dot(e, v, preferred_element_type=jnp.float32)
    den = oa[:, 3:4]
    o = oa[:, :3] * jnp.where(den > 0, 1.0 / den, 0.0)

    w1f = jnp.dot(wo_ref[...], w1_ref[...], preferred_element_type=jnp.float32)
    b1f = jnp.dot(bo_ref[...], w1_ref[...],
                  preferred_element_type=jnp.float32) + b1_ref[...]
    h = jnp.maximum(
        jnp.dot(o, w1f, preferred_element_type=jnp.float32) + b1f, 0.0)
    out_ref[...] = (
        jnp.dot(h, w2_ref[...], preferred_element_type=jnp.float32)
        + b2_ref[...])


def kernel(x, Wq, bq, Wk, bk, Wv, bv, Wo, bo, W1, b1, W2, b2, labels):
    B, N, D = x.shape
    H = W1.shape[1]
    scale = float(np.log2(np.e) / np.sqrt(D))
    lab = labels.astype(jnp.int32).reshape(B, 1, N)

    wspec = lambda shape: pl.BlockSpec(shape, lambda b: (0, 0))
    out = pl.pallas_call(
        functools.partial(_fused_kernel, scale=scale, k_out=_K_OUT),
        grid=(B,),
        in_specs=[
            pl.BlockSpec((None, N, D), lambda b: (b, 0, 0)),
            pl.BlockSpec((None, 1, N), lambda b: (b, 0, 0)),
            wspec((D, D)), wspec((1, D)),
            wspec((D, D)), wspec((1, D)),
            wspec((D, D)), wspec((1, D)),
            wspec((D, D)), wspec((1, D)),
            wspec((D, H)), wspec((1, H)),
            wspec((H, D)), wspec((1, D)),
        ],
        out_specs=pl.BlockSpec((None, _K_OUT, D), lambda b: (b, 0, 0)),
        out_shape=jax.ShapeDtypeStruct((B, _K_OUT, D), jnp.float32),
        compiler_params=pltpu.CompilerParams(
            dimension_semantics=("arbitrary",)),
    )(x, lab, Wq, bq.reshape(1, D), Wk, bk.reshape(1, D), Wv,
      bv.reshape(1, D), Wo, bo.reshape(1, D), W1, b1.reshape(1, H),
      W2, b2.reshape(1, D))
    return out
